# Initial kernel scaffold; baseline (speedup 1.0000x reference)
import jax
import jax.numpy as jnp
from jax.experimental import pallas as pl


def kernel(inputs, gate_w, expert_w):
    raise NotImplementedError("write your pallas kernel here")


# speedup vs baseline: 1.0853x; 1.0853x over previous
import jax
import jax.numpy as jnp
from jax.experimental import pallas as pl
from jax.experimental.pallas import tpu as pltpu

_O_BLK = 512


def _moe_step(x_ref, gw_ref, w_ref, o_ref):
    x = x_ref[...]
    logits = jnp.dot(x, gw_ref[...], preferred_element_type=jnp.float32)
    v0 = jnp.max(logits, axis=1, keepdims=True)
    e_idx = jax.lax.broadcasted_iota(jnp.int32, logits.shape, 1)
    sel0 = jnp.min(jnp.where(logits == v0, e_idx, logits.shape[1]),
                   axis=1, keepdims=True)
    masked = jnp.where(e_idx == sel0, -jnp.inf, logits)
    v1 = jnp.max(masked, axis=1, keepdims=True)
    w0 = jax.nn.sigmoid(v0 - v1)
    s = jnp.where(sel0 != 0, w0, 0.0)
    o_ref[0] = jnp.dot(x * s, w_ref[0], preferred_element_type=jnp.float32)


def kernel(inputs, gate_w, expert_w):
    B, S, H = inputs.shape
    D, _, O = expert_w.shape
    x = inputs.reshape(B * S, H)
    out = pl.pallas_call(
        _moe_step,
        grid=(D, O // _O_BLK),
        in_specs=[
            pl.BlockSpec((B * S, H), lambda i, j: (0, 0)),
            pl.BlockSpec((H, gate_w.shape[1]), lambda i, j: (0, 0)),
            pl.BlockSpec((1, H, _O_BLK), lambda i, j: (i, 0, j)),
        ],
        out_specs=pl.BlockSpec((1, B * S, _O_BLK), lambda i, j: (i, 0, j)),
        out_shape=jax.ShapeDtypeStruct((D, B * S, O), jnp.float32),
        compiler_params=pltpu.CompilerParams(
            dimension_semantics=("parallel", "parallel")),
    )(x, gate_w, expert_w)
    return out.reshape(D, B, S, 1, O)
